# baseline (speedup 1.0000x reference)
import functools

import jax
import jax.numpy as jnp
from jax import lax
from jax.experimental import pallas as pl
from jax.experimental.pallas import tpu as pltpu
from jax.experimental.pallas import tpu_sc as plsc

NUM_EMBEDDINGS = 1000000
EMBEDDING_DIM = 64
TP_WORLD_SIZE = 2
NUM_EMB_PER_PART = NUM_EMBEDDINGS // TP_WORLD_SIZE
VOCAB_START = 0
VOCAB_END = NUM_EMB_PER_PART
NUM_TOKENS = 16384

NC = 2
NS = 16
NW = NC * NS
BPW = NUM_TOKENS // NW
GCHUNK = 128
NCHUNK = BPW // GCHUNK

_mesh = plsc.VectorSubcoreMesh(core_axis_name="c", subcore_axis_name="s")


@functools.partial(
    pl.kernel,
    mesh=_mesh,
    out_type=jax.ShapeDtypeStruct((NUM_TOKENS, EMBEDDING_DIM), jnp.float32),
    scratch_types=[
        pltpu.VMEM((NCHUNK, GCHUNK), jnp.int32),
        pltpu.VMEM((BPW,), jnp.float32),
        pltpu.VMEM((BPW, EMBEDDING_DIM), jnp.float32),
        pltpu.SemaphoreType.DMA,
    ],
    compiler_params=pltpu.CompilerParams(
        use_tc_tiling_on_sc=False, needs_layout_passes=False
    ),
)
def _emb_kernel(x_hbm, w_hbm, out_hbm, idx_v, fmask_v, rows_v, sem):
    wid = lax.axis_index("s") * NC + lax.axis_index("c")
    base = wid * BPW

    for c in range(NCHUNK):
        pltpu.sync_copy(x_hbm.at[pl.ds(base + c * GCHUNK, GCHUNK)], idx_v.at[c])

    ones_f = jnp.full((16,), 1.0, jnp.float32)
    zeros_f = jnp.full((16,), 0.0, jnp.float32)
    zeros_i = jnp.full((16,), 0, jnp.int32)
    span = jnp.full((16,), VOCAB_END - VOCAB_START, jnp.uint32)

    copies = []
    for c in range(NCHUNK):
        for g in range(GCHUNK // 16):
            iv = idx_v[c, pl.ds(g * 16, 16)]
            rel = iv - VOCAB_START
            m = plsc.bitcast(rel, jnp.uint32) < span
            idx_v[c, pl.ds(g * 16, 16)] = jnp.where(m, rel, zeros_i)
            fmask_v[pl.ds(c * GCHUNK + g * 16, 16)] = jnp.where(m, ones_f, zeros_f)
        copies.append(
            pltpu.async_copy(
                w_hbm.at[idx_v.at[c]],
                rows_v.at[pl.ds(c * GCHUNK, GCHUNK)],
                sem,
            )
        )
    for cp in copies:
        cp.wait()

    def mul_body(t, _):
        bc = plsc.load_gather(fmask_v, [jnp.full((16,), 0, jnp.int32) + t])
        for j in range(EMBEDDING_DIM // 16):
            seg = rows_v[t, pl.ds(j * 16, 16)]
            rows_v[t, pl.ds(j * 16, 16)] = seg * bc
        return 0

    lax.fori_loop(0, BPW, mul_body, 0)

    pltpu.sync_copy(rows_v, out_hbm.at[pl.ds(base, BPW)])


def kernel(x, weight):
    return _emb_kernel(x.astype(jnp.int32), weight)
